# baseline (speedup 1.0000x reference)
import jax, jax.numpy as jnp
from jax.experimental import pallas as pl

N = 10000

def _copy_body(i_ref, o_ref):
    o_ref[...] = i_ref[...]

def kernel(x, edge_index, W1a, b1a, W1b, b1b, g1, bt1, m1, v1,
           W2a, b2a, W2b, b2b, g2, bt2, m2, v2,
           W3a, b3a, W3b, b3b, g3, bt3, m3, v3, fcW, fcb, outW, outb):
    src = edge_index[0]; dst = edge_index[1]
    def agg(h):
        return jax.ops.segment_sum(h[src], dst, num_segments=N)
    def bn(h, g, b, m, v):
        return (h - m) / jnp.sqrt(v + 1e-5) * g + b
    h = x + agg(x)
    h = jnp.maximum(h @ W1a + b1a, 0.0) @ W1b + b1b
    h = bn(jnp.maximum(h, 0.0), g1, bt1, m1, v1)
    h = h + agg(h)
    h = jnp.maximum(h @ W2a + b2a, 0.0) @ W2b + b2b
    h = bn(jnp.maximum(h, 0.0), g2, bt2, m2, v2)
    h = h + agg(h)
    h = jnp.maximum(h @ W3a + b3a, 0.0) @ W3b + b3b
    h = bn(jnp.maximum(h, 0.0), g3, bt3, m3, v3)
    h = jnp.maximum(h @ fcW + fcb, 0.0)
    logits = h @ outW + outb
    out = jax.nn.log_softmax(logits, axis=-1)
    return pl.pallas_call(_copy_body,
        out_shape=jax.ShapeDtypeStruct(out.shape, out.dtype))(out)


# speedup vs baseline: 2.6468x; 2.6468x over previous
import functools

import jax
import jax.numpy as jnp
from jax import lax
from jax.experimental import pallas as pl
from jax.experimental.pallas import tpu as pltpu
from jax.experimental.pallas import tpu_sc as plsc
from jax.scipy.linalg import block_diag

N = 10000
E = 320000
D = 128
H = 32
C = 10

NC = 2
NS = 16

NP = 10112
NPQ = NP // 4
QR = NP // 4
RB = 2048
EPADH = 163840
EPAD = EPADH * NC
ESL = EPADH // 4
NRT = ESL // RB
CH = 128
NCHK = RB // CH

_mesh = plsc.VectorSubcoreMesh(core_axis_name="c", subcore_axis_name="s")


@functools.partial(
    pl.kernel,
    out_type=jax.ShapeDtypeStruct((NC, 4, NP * H), jnp.float32),
    mesh=_mesh,
    scratch_types=[
        pltpu.VMEM((RB,), jnp.int32),
        pltpu.VMEM((RB,), jnp.int32),
        pltpu.VMEM((RB,), jnp.int32),
        pltpu.VMEM((CH, D), jnp.float32),
        pltpu.VMEM((CH, D), jnp.float32),
        pltpu.VMEM(((QR + 8) * H,), jnp.float32),
        pltpu.SemaphoreType.DMA,
        pltpu.SemaphoreType.DMA,
    ],
)
def _segsum2(h_hbm, src_hbm, dst_hbm, out_hbm,
             sq, lvb, sbuf, rows0, rows1, acc, sem0, sem1):
    cid = lax.axis_index("c")
    sid = lax.axis_index("s")
    g = lax.shift_right_logical(sid, 2)
    j = sid & 3
    lo = g * QR
    ebase = cid * EPADH + j * ESL

    zeros16 = jnp.zeros((16,), jnp.float32)

    def zrow(r, carry):
        acc[pl.ds(r * 32, 16)] = zeros16
        acc[pl.ds(r * 32 + 16, 16)] = zeros16
        return carry

    lax.fori_loop(0, QR + 8, zrow, 0)

    def start_gather(k, buf, sem):
        pltpu.async_copy(h_hbm.at[sq.at[pl.ds(k * CH, CH)]], buf, sem)

    def wait_rows(buf, sem):
        pltpu.make_async_copy(h_hbm.at[pl.ds(0, CH)], buf, sem).wait()

    def accum(k, buf):
        def egroup(g2, carry3):
            lv16 = lvb[pl.ds(k * CH + g2 * 16, 16)]
            sv16 = sbuf[pl.ds(k * CH + g2 * 16, 16)]
            of16 = lax.shift_left(sv16 & 3, 5)
            for e2 in range(16):
                e = g2 * 16 + e2
                base = lv16[e2] * 32
                off = of16[e2]
                plsc.addupdate(acc.at[pl.ds(base, 16)],
                               buf[e, pl.ds(off, 16)])
                plsc.addupdate(acc.at[pl.ds(base + 16, 16)],
                               buf[e, pl.ds(off + 16, 16)])
            return carry3

        lax.fori_loop(0, CH // 16, egroup, 0)

    def round_body(r, carry):
        pltpu.sync_copy(src_hbm.at[pl.ds(ebase + r * RB, RB)], sbuf)
        pltpu.sync_copy(dst_hbm.at[pl.ds(ebase + r * RB, RB)], lvb)

        def prep(p, carry1):
            sv = sbuf[pl.ds(p * 16, 16)]
            dv = lvb[pl.ds(p * 16, 16)]
            lv = dv - lo
            m = (lv >= 0) & (lv < QR)
            sq[pl.ds(p * 16, 16)] = lax.shift_right_logical(sv, 2)
            lvb[pl.ds(p * 16, 16)] = jnp.where(m, lv, QR)
            return carry1

        lax.fori_loop(0, RB // 16, prep, 0)

        start_gather(0, rows0, sem0)

        def pair(i, carry2):
            k = 2 * i
            start_gather(k + 1, rows1, sem1)
            wait_rows(rows0, sem0)
            accum(k, rows0)
            start_gather((k + 2) % NCHK, rows0, sem0)
            wait_rows(rows1, sem1)
            accum(k + 1, rows1)
            return carry2

        lax.fori_loop(0, NCHK // 2, pair, 0)
        wait_rows(rows0, sem0)
        return carry

    lax.fori_loop(0, NRT, round_body, 0)

    pltpu.sync_copy(acc.at[pl.ds(0, QR * H)],
                    out_hbm.at[cid, j, pl.ds(lo * H, QR * H)])


BRQ = 632


def _proj_body(x_ref, w_ref, o_ref):
    o_ref[...] = jnp.dot(x_ref[...], w_ref[...],
                         preferred_element_type=jnp.float32)


def _mlp1_body(y_ref, a_ref, b1a_ref, w1b_ref, b1b_ref, s1_ref, t1_ref, o_ref):
    u = jnp.maximum(y_ref[...] + jnp.sum(a_ref[...], axis=0) + b1a_ref[...], 0.0)
    z = jnp.dot(u, w1b_ref[...], preferred_element_type=jnp.float32) + b1b_ref[...]
    o_ref[...] = jnp.maximum(z, 0.0) * s1_ref[...] + t1_ref[...]


def _mlp_body(h_ref, a_ref, wa_ref, ba_ref, wb_ref, bb_ref, s_ref, t_ref, o_ref):
    g = h_ref[...] + jnp.sum(a_ref[...], axis=0)
    u = jnp.maximum(jnp.dot(g, wa_ref[...], preferred_element_type=jnp.float32)
                    + ba_ref[...], 0.0)
    z = jnp.dot(u, wb_ref[...], preferred_element_type=jnp.float32) + bb_ref[...]
    o_ref[...] = jnp.maximum(z, 0.0) * s_ref[...] + t_ref[...]


def _head_body(h_ref, a_ref, w3a_ref, b3a_ref, w3b_ref, b3b_ref, s3_ref,
               t3_ref, fcw_ref, fcb_ref, outw_ref, outb_ref, o_ref):
    g = h_ref[...] + jnp.sum(a_ref[...], axis=0)
    u = jnp.maximum(jnp.dot(g, w3a_ref[...], preferred_element_type=jnp.float32)
                    + b3a_ref[...], 0.0)
    z = jnp.dot(u, w3b_ref[...], preferred_element_type=jnp.float32) + b3b_ref[...]
    h3 = jnp.maximum(z, 0.0) * s3_ref[...] + t3_ref[...]
    h4 = jnp.maximum(jnp.dot(h3, fcw_ref[...], preferred_element_type=jnp.float32)
                     + fcb_ref[...], 0.0)
    o_ref[...] = jnp.dot(h4, outw_ref[...], preferred_element_type=jnp.float32) \
        + outb_ref[...]


def _lsm_body(l_ref, o_ref):
    logits = l_ref[...]
    m = jnp.max(logits, axis=-1, keepdims=True)
    lse = m + jnp.log(jnp.sum(jnp.exp(logits - m), axis=-1, keepdims=True))
    o_ref[...] = logits - lse


_vecq = lambda: pl.BlockSpec((1, 4 * H), lambda i: (0, 0))
_matq = lambda: pl.BlockSpec((4 * H, 4 * H), lambda i: (0, 0))
_rowsq = lambda: pl.BlockSpec((BRQ, D), lambda i: (i, 0))
_partsq = lambda: pl.BlockSpec((NC * 4, BRQ, D), lambda i: (0, i, 0))


def kernel(x, edge_index, W1a, b1a, W1b, b1b, g1, bt1, m1, v1,
           W2a, b2a, W2b, b2b, g2, bt2, m2, v2,
           W3a, b3a, W3b, b3b, g3, bt3, m3, v3, fcW, fcb, outW, outb):
    f32 = jnp.float32

    pad = N + (jnp.arange(EPAD - E, dtype=jnp.int32) % (NP - N))
    src = jnp.concatenate([edge_index[0], pad])
    dst = jnp.concatenate([edge_index[1], pad])
    blk = lambda W: block_diag(W, W, W, W)
    til = lambda v: jnp.tile(v, 4).reshape(1, 4 * H)
    xq = jnp.pad(x, ((0, NP - N), (0, 0))).reshape(NPQ, 4 * D)
    W1aq = block_diag(W1a, W1a, W1a, W1a)

    def bn_coeffs(g, bt, m, v):
        s = g / jnp.sqrt(v + 1e-5)
        return til(s), til(bt - m * s)

    s1, t1 = bn_coeffs(g1, bt1, m1, v1)
    s2, t2 = bn_coeffs(g2, bt2, m2, v2)
    s3, t3 = bn_coeffs(g3, bt3, m3, v3)

    def agg(hq):
        a = _segsum2(hq, src, dst)
        return a.reshape(NC * 4, NPQ, 4 * H)

    y = pl.pallas_call(
        _proj_body,
        grid=(NPQ // BRQ,),
        in_specs=[pl.BlockSpec((BRQ, 4 * D), lambda i: (i, 0)),
                  pl.BlockSpec((4 * D, 4 * H), lambda i: (0, 0))],
        out_specs=pl.BlockSpec((BRQ, D), lambda i: (i, 0)),
        out_shape=jax.ShapeDtypeStruct((NPQ, D), f32),
    )(xq, W1aq)
    a = agg(y)
    h1 = pl.pallas_call(
        _mlp1_body,
        grid=(NPQ // BRQ,),
        in_specs=[_rowsq(), _partsq(), _vecq(), _matq(), _vecq(), _vecq(),
                  _vecq()],
        out_specs=pl.BlockSpec((BRQ, D), lambda i: (i, 0)),
        out_shape=jax.ShapeDtypeStruct((NPQ, D), f32),
    )(y, a, til(b1a), blk(W1b), til(b1b), s1, t1)

    def mid_layer(h, Wa, ba, Wb, bb, s, t):
        a = agg(h)
        return pl.pallas_call(
            _mlp_body,
            grid=(NPQ // BRQ,),
            in_specs=[_rowsq(), _partsq(), _matq(), _vecq(), _matq(), _vecq(),
                      _vecq(), _vecq()],
            out_specs=pl.BlockSpec((BRQ, D), lambda i: (i, 0)),
            out_shape=jax.ShapeDtypeStruct((NPQ, D), f32),
        )(h, a, blk(Wa), til(ba), blk(Wb), til(bb), s, t)

    h2 = mid_layer(h1, W2a, b2a, W2b, b2b, s2, t2)

    a = agg(h2)
    logits_q = pl.pallas_call(
        _head_body,
        grid=(NPQ // BRQ,),
        in_specs=[_rowsq(), _partsq(), _matq(), _vecq(), _matq(), _vecq(),
                  _vecq(), _vecq(), _matq(), _vecq(),
                  pl.BlockSpec((4 * H, 4 * C), lambda i: (0, 0)),
                  pl.BlockSpec((1, 4 * C), lambda i: (0, 0))],
        out_specs=pl.BlockSpec((BRQ, 4 * C), lambda i: (i, 0)),
        out_shape=jax.ShapeDtypeStruct((NPQ, 4 * C), f32),
    )(h2, a, blk(W3a), til(b3a), blk(W3b), til(b3b), s3, t3,
      blk(fcW), til(fcb), block_diag(outW, outW, outW, outW),
      jnp.tile(outb, 4).reshape(1, 4 * C))

    logits = logits_q.reshape(NP, C)
    out = pl.pallas_call(
        _lsm_body,
        grid=(4,),
        in_specs=[pl.BlockSpec((NP // 4, C), lambda i: (i, 0))],
        out_specs=pl.BlockSpec((NP // 4, C), lambda i: (i, 0)),
        out_shape=jax.ShapeDtypeStruct((NP, C), f32),
    )(logits)
    return out[:N]
